# baseline (speedup 1.0000x reference)
import functools

import jax
import jax.numpy as jnp
from jax.experimental import pallas as pl
from jax.experimental.pallas import tpu as pltpu


def _body(t_ref, x_ref, w1_ref, b1_ref, w2_ref, b2_ref, wp_ref, g1_ref,
          c_ref, o_ref, y_scr, mu_scr, rs_scr, *, n_types, out_d, hidden):
    i = pl.program_id(0)
    p = jax.lax.rem(i, 2)
    q = 1 - p

    x = x_ref[...].astype(jnp.bfloat16)
    h = jnp.dot(x, w1_ref[...], preferred_element_type=jnp.float32)
    h = jnp.maximum(h + b1_ref[...], 0.0)
    h = jnp.dot(h.astype(jnp.bfloat16), w2_ref[...],
                preferred_element_type=jnp.float32)
    h = h + b2_ref[...]
    y_scr[p] = jnp.dot(h.astype(jnp.bfloat16), wp_ref[...],
                       preferred_element_type=jnp.float32)
    inv_h = 1.0 / hidden
    mu = jnp.sum(h, axis=-1, keepdims=True) * inv_h
    m2 = jnp.sum(h * h, axis=-1, keepdims=True) * inv_h
    mu_scr[p] = mu
    rs_scr[p] = jax.lax.rsqrt(jnp.maximum(m2 - mu * mu, 0.0) + 1e-5)

    y = y_scr[q]
    mu1 = mu_scr[q]
    rstd = rs_scr[q]
    t = t_ref[...]
    y_sel = None
    g_sel = None
    c_sel = None
    for tt in range(n_types):
        mask = t == tt
        sl = slice(tt * out_d, (tt + 1) * out_d)
        ys = jnp.where(mask, y[:, sl], 0.0)
        gs = jnp.where(mask, g1_ref[:, sl], 0.0)
        cs = jnp.where(mask, c_ref[:, sl], 0.0)
        if y_sel is None:
            y_sel, g_sel, c_sel = ys, gs, cs
        else:
            y_sel, g_sel, c_sel = y_sel + ys, g_sel + gs, c_sel + cs
    o_ref[...] = rstd * y_sel - (rstd * mu1) * g_sel + c_sel


def kernel(node_latent, node_types, w1, b1, w2, b2, ln_gamma, ln_beta,
           head_w, head_b, *, interpret=False):
    n, latent = node_latent.shape
    hidden = w1.shape[1]
    out_d = head_w.shape[2]
    n_types = head_w.shape[0]
    bm = 1000
    g = n // bm

    t2 = node_types.reshape(n, 1)
    b1r = b1.reshape(1, hidden)
    b2r = b2.reshape(1, hidden)
    w1b = w1.astype(jnp.bfloat16)
    w2b = w2.astype(jnp.bfloat16)
    w_cat = head_w.transpose(1, 0, 2).reshape(hidden, n_types * out_d)
    wp = (ln_gamma[:, None] * w_cat).astype(jnp.bfloat16)
    g1 = (ln_gamma @ w_cat).reshape(1, n_types * out_d)
    c_all = (ln_beta @ w_cat).reshape(1, n_types * out_d) \
        + head_b.reshape(1, n_types * out_d)

    def lag(i):
        return jnp.maximum(i - 1, 0)

    def clamp(i):
        return jnp.minimum(i, g - 1)

    return pl.pallas_call(
        functools.partial(_body, n_types=n_types, out_d=out_d, hidden=hidden),
        out_shape=jax.ShapeDtypeStruct((n, out_d), jnp.float32),
        grid=(g + 1,),
        in_specs=[
            pl.BlockSpec((bm, 1), lambda i: (lag(i), 0)),
            pl.BlockSpec((bm, latent), lambda i: (clamp(i), 0)),
            pl.BlockSpec((latent, hidden), lambda i: (0, 0)),
            pl.BlockSpec((1, hidden), lambda i: (0, 0)),
            pl.BlockSpec((hidden, hidden), lambda i: (0, 0)),
            pl.BlockSpec((1, hidden), lambda i: (0, 0)),
            pl.BlockSpec((hidden, n_types * out_d), lambda i: (0, 0)),
            pl.BlockSpec((1, n_types * out_d), lambda i: (0, 0)),
            pl.BlockSpec((1, n_types * out_d), lambda i: (0, 0)),
        ],
        out_specs=pl.BlockSpec((bm, out_d), lambda i: (lag(i), 0)),
        scratch_shapes=[
            pltpu.VMEM((2, bm, n_types * out_d), jnp.float32),
            pltpu.VMEM((2, bm, 1), jnp.float32),
            pltpu.VMEM((2, bm, 1), jnp.float32),
        ],
        compiler_params=pltpu.CompilerParams(
            dimension_semantics=("arbitrary",),
            vmem_limit_bytes=56 * 1024 * 1024,
        ),
        name="adaptive_decoder",
        interpret=interpret,
    )(t2, node_latent, w1b, b1r, w2b, b2r, wp, g1, c_all)


# speedup vs baseline: 1.0696x; 1.0696x over previous
import functools

import jax
import jax.numpy as jnp
from jax.experimental import pallas as pl
from jax.experimental.pallas import tpu as pltpu


def _body(t_ref, x_ref, w1_ref, b1_ref, w2_ref, b2_ref, wp_ref, g1_ref,
          c_ref, o_ref, y_scr, mu_scr, rs_scr, *, n_types, out_d, hidden):
    y = y_scr[...]
    mu1 = mu_scr[...]
    rstd = rs_scr[...]
    t = t_ref[...]
    y_sel = None
    g_sel = None
    c_sel = None
    for tt in range(n_types):
        mask = t == tt
        sl = slice(tt * out_d, (tt + 1) * out_d)
        ys = jnp.where(mask, y[:, sl], 0.0)
        gs = jnp.where(mask, g1_ref[:, sl], 0.0)
        cs = jnp.where(mask, c_ref[:, sl], 0.0)
        if y_sel is None:
            y_sel, g_sel, c_sel = ys, gs, cs
        else:
            y_sel, g_sel, c_sel = y_sel + ys, g_sel + gs, c_sel + cs
    o_ref[...] = rstd * y_sel - (rstd * mu1) * g_sel + c_sel

    x = x_ref[...].astype(jnp.bfloat16)
    h = jnp.dot(x, w1_ref[...], preferred_element_type=jnp.float32)
    h = jnp.maximum(h + b1_ref[...], 0.0)
    h = jnp.dot(h.astype(jnp.bfloat16), w2_ref[...],
                preferred_element_type=jnp.float32)
    h = h + b2_ref[...]
    y_scr[...] = jnp.dot(h.astype(jnp.bfloat16), wp_ref[...],
                         preferred_element_type=jnp.float32)
    inv_h = 1.0 / hidden
    mu = jnp.sum(h, axis=-1, keepdims=True) * inv_h
    m2 = jnp.sum(h * h, axis=-1, keepdims=True) * inv_h
    mu_scr[...] = mu
    rs_scr[...] = jax.lax.rsqrt(jnp.maximum(m2 - mu * mu, 0.0) + 1e-5)


def kernel(node_latent, node_types, w1, b1, w2, b2, ln_gamma, ln_beta,
           head_w, head_b, *, interpret=False):
    n, latent = node_latent.shape
    hidden = w1.shape[1]
    out_d = head_w.shape[2]
    n_types = head_w.shape[0]
    bm = 1000
    g = n // bm

    t2 = node_types.reshape(n, 1)
    b1r = b1.reshape(1, hidden)
    b2r = b2.reshape(1, hidden)
    w1b = w1.astype(jnp.bfloat16)
    w2b = w2.astype(jnp.bfloat16)
    w_cat = head_w.transpose(1, 0, 2).reshape(hidden, n_types * out_d)
    wp = (ln_gamma[:, None] * w_cat).astype(jnp.bfloat16)
    g1 = (ln_gamma @ w_cat).reshape(1, n_types * out_d)
    c_all = (ln_beta @ w_cat).reshape(1, n_types * out_d) \
        + head_b.reshape(1, n_types * out_d)

    def lag(i):
        return jnp.maximum(i - 1, 0)

    def clamp(i):
        return jnp.minimum(i, g - 1)

    return pl.pallas_call(
        functools.partial(_body, n_types=n_types, out_d=out_d, hidden=hidden),
        out_shape=jax.ShapeDtypeStruct((n, out_d), jnp.float32),
        grid=(g + 1,),
        in_specs=[
            pl.BlockSpec((bm, 1), lambda i: (lag(i), 0)),
            pl.BlockSpec((bm, latent), lambda i: (clamp(i), 0)),
            pl.BlockSpec((latent, hidden), lambda i: (0, 0)),
            pl.BlockSpec((1, hidden), lambda i: (0, 0)),
            pl.BlockSpec((hidden, hidden), lambda i: (0, 0)),
            pl.BlockSpec((1, hidden), lambda i: (0, 0)),
            pl.BlockSpec((hidden, n_types * out_d), lambda i: (0, 0)),
            pl.BlockSpec((1, n_types * out_d), lambda i: (0, 0)),
            pl.BlockSpec((1, n_types * out_d), lambda i: (0, 0)),
        ],
        out_specs=pl.BlockSpec((bm, out_d), lambda i: (lag(i), 0)),
        scratch_shapes=[
            pltpu.VMEM((bm, n_types * out_d), jnp.float32),
            pltpu.VMEM((bm, 1), jnp.float32),
            pltpu.VMEM((bm, 1), jnp.float32),
        ],
        compiler_params=pltpu.CompilerParams(
            dimension_semantics=("arbitrary",),
            vmem_limit_bytes=56 * 1024 * 1024,
        ),
        name="adaptive_decoder",
        interpret=interpret,
    )(t2, node_latent, w1b, b1r, w2b, b2r, wp, g1, c_all)


# speedup vs baseline: 1.0741x; 1.0042x over previous
import functools

import jax
import jax.numpy as jnp
from jax.experimental import pallas as pl
from jax.experimental.pallas import tpu as pltpu


def _body(t_ref, x_ref, w1_ref, b1_ref, w2_ref, b2_ref, wp_ref, g1_ref,
          c_ref, o_ref, *, n_types, out_d, hidden, n_split):
    bm = x_ref.shape[0]
    hm = bm // n_split
    for s in range(n_split):
        rows = slice(s * hm, (s + 1) * hm)
        x = x_ref[rows, :].astype(jnp.bfloat16)
        h = jnp.dot(x, w1_ref[...], preferred_element_type=jnp.float32)
        h = jnp.maximum(h + b1_ref[...], 0.0)
        h = jnp.dot(h.astype(jnp.bfloat16), w2_ref[...],
                    preferred_element_type=jnp.float32)
        h = h + b2_ref[...]
        y = jnp.dot(h.astype(jnp.bfloat16), wp_ref[...],
                    preferred_element_type=jnp.float32)
        inv_h = 1.0 / hidden
        mu = jnp.sum(h, axis=-1, keepdims=True) * inv_h
        m2 = jnp.sum(h * h, axis=-1, keepdims=True) * inv_h
        rstd = jax.lax.rsqrt(jnp.maximum(m2 - mu * mu, 0.0) + 1e-5)
        t = t_ref[rows, :]
        y_sel = None
        g_sel = None
        c_sel = None
        for tt in range(n_types):
            mask = t == tt
            sl = slice(tt * out_d, (tt + 1) * out_d)
            ys = jnp.where(mask, y[:, sl], 0.0)
            gs = jnp.where(mask, g1_ref[:, sl], 0.0)
            cs = jnp.where(mask, c_ref[:, sl], 0.0)
            if y_sel is None:
                y_sel, g_sel, c_sel = ys, gs, cs
            else:
                y_sel, g_sel, c_sel = y_sel + ys, g_sel + gs, c_sel + cs
        o_ref[rows, :] = rstd * y_sel - (rstd * mu) * g_sel + c_sel


def kernel(node_latent, node_types, w1, b1, w2, b2, ln_gamma, ln_beta,
           head_w, head_b, *, interpret=False, bm=2000, n_split=1,
           semantics="parallel"):
    n, latent = node_latent.shape
    hidden = w1.shape[1]
    out_d = head_w.shape[2]
    n_types = head_w.shape[0]
    nb = n // bm
    if semantics == "core_parallel":
        grid = (2, nb // 2)
        half = nb // 2

        def rowmap(c, j):
            return (c * half + j, 0)

        def zmap(c, j):
            return (0, 0)
    else:
        grid = (nb,)

        def rowmap(i):
            return (i, 0)

        def zmap(i):
            return (0, 0)

    t2 = node_types.reshape(n, 1)
    b1r = b1.reshape(1, hidden)
    b2r = b2.reshape(1, hidden)
    w1b = w1.astype(jnp.bfloat16)
    w2b = w2.astype(jnp.bfloat16)
    w_cat = head_w.transpose(1, 0, 2).reshape(hidden, n_types * out_d)
    wp = (ln_gamma[:, None] * w_cat).astype(jnp.bfloat16)
    g1 = (ln_gamma @ w_cat).reshape(1, n_types * out_d)
    c_all = (ln_beta @ w_cat).reshape(1, n_types * out_d) \
        + head_b.reshape(1, n_types * out_d)

    return pl.pallas_call(
        functools.partial(_body, n_types=n_types, out_d=out_d, hidden=hidden,
                          n_split=n_split),
        out_shape=jax.ShapeDtypeStruct((n, out_d), jnp.float32),
        grid=grid,
        in_specs=[
            pl.BlockSpec((bm, 1), rowmap),
            pl.BlockSpec((bm, latent), rowmap),
            pl.BlockSpec((latent, hidden), zmap),
            pl.BlockSpec((1, hidden), zmap),
            pl.BlockSpec((hidden, hidden), zmap),
            pl.BlockSpec((1, hidden), zmap),
            pl.BlockSpec((hidden, n_types * out_d), zmap),
            pl.BlockSpec((1, n_types * out_d), zmap),
            pl.BlockSpec((1, n_types * out_d), zmap),
        ],
        out_specs=pl.BlockSpec((bm, out_d), rowmap),
        compiler_params=pltpu.CompilerParams(
            dimension_semantics=(("core_parallel", "arbitrary")
                                 if semantics == "core_parallel"
                                 else (semantics,)),
            vmem_limit_bytes=56 * 1024 * 1024,
        ),
        name="adaptive_decoder",
        interpret=interpret,
    )(t2, node_latent, w1b, b1r, w2b, b2r, wp, g1, c_all)


# speedup vs baseline: 1.2357x; 1.1505x over previous
import functools

import jax
import jax.numpy as jnp
from jax.experimental import pallas as pl
from jax.experimental.pallas import tpu as pltpu


def _body(t_ref, x_ref, w1_ref, b1_ref, w2_ref, b2_ref, wp_ref, gc_ref,
          o_ref, *, n_types, out_d, hidden):
    x = x_ref[...].astype(jnp.bfloat16)
    h = jnp.dot(x, w1_ref[...], preferred_element_type=jnp.float32)
    h = jnp.maximum(h + b1_ref[...], 0.0)
    h = jnp.dot(h.astype(jnp.bfloat16), w2_ref[...],
                preferred_element_type=jnp.float32)
    h = h + b2_ref[...]
    y = jnp.dot(h.astype(jnp.bfloat16), wp_ref[...],
                preferred_element_type=jnp.float32)
    inv_h = 1.0 / hidden
    mu = jnp.sum(h, axis=-1, keepdims=True) * inv_h
    m2 = jnp.sum(h * h, axis=-1, keepdims=True) * inv_h
    rstd = jax.lax.rsqrt(jnp.maximum(m2 - mu * mu, 0.0) + 1e-5)

    t = t_ref[...]
    bm = t.shape[0]
    lanes = jax.lax.broadcasted_iota(jnp.int32, (bm, 128), 1)
    onehot = (lanes == t).astype(jnp.bfloat16)
    corr = jnp.dot(onehot, gc_ref[...],
                   preferred_element_type=jnp.float32)
    g_sel = corr[:, :out_d]
    c_sel = corr[:, out_d:]

    y_sel = y[:, (n_types - 1) * out_d:]
    for tt in range(n_types - 2, -1, -1):
        y_sel = jnp.where(t == tt, y[:, tt * out_d:(tt + 1) * out_d], y_sel)
    o_ref[...] = rstd * y_sel - (rstd * mu) * g_sel + c_sel


def kernel(node_latent, node_types, w1, b1, w2, b2, ln_gamma, ln_beta,
           head_w, head_b, *, interpret=False, bm=1000):
    n, latent = node_latent.shape
    hidden = w1.shape[1]
    out_d = head_w.shape[2]
    n_types = head_w.shape[0]
    grid = (n // bm,)

    t2 = node_types.reshape(n, 1)
    b1r = b1.reshape(1, hidden)
    b2r = b2.reshape(1, hidden)
    w1b = w1.astype(jnp.bfloat16)
    w2b = w2.astype(jnp.bfloat16)
    w_cat = head_w.transpose(1, 0, 2).reshape(hidden, n_types * out_d)
    wp = (ln_gamma[:, None] * w_cat).astype(jnp.bfloat16)
    g1 = (ln_gamma @ w_cat).reshape(n_types, out_d)
    c_all = (ln_beta @ w_cat).reshape(n_types, out_d) + head_b
    gc = jnp.zeros((128, 2 * out_d), jnp.float32)
    gc = gc.at[:n_types, :out_d].set(g1).at[:n_types, out_d:].set(c_all)
    gcb = gc.astype(jnp.bfloat16)

    return pl.pallas_call(
        functools.partial(_body, n_types=n_types, out_d=out_d, hidden=hidden),
        out_shape=jax.ShapeDtypeStruct((n, out_d), jnp.float32),
        grid=grid,
        in_specs=[
            pl.BlockSpec((bm, 1), lambda i: (i, 0)),
            pl.BlockSpec((bm, latent), lambda i: (i, 0)),
            pl.BlockSpec((latent, hidden), lambda i: (0, 0)),
            pl.BlockSpec((1, hidden), lambda i: (0, 0)),
            pl.BlockSpec((hidden, hidden), lambda i: (0, 0)),
            pl.BlockSpec((1, hidden), lambda i: (0, 0)),
            pl.BlockSpec((hidden, n_types * out_d), lambda i: (0, 0)),
            pl.BlockSpec((128, 2 * out_d), lambda i: (0, 0)),
        ],
        out_specs=pl.BlockSpec((bm, out_d), lambda i: (i, 0)),
        compiler_params=pltpu.CompilerParams(
            dimension_semantics=("parallel",),
            vmem_limit_bytes=56 * 1024 * 1024,
        ),
        name="adaptive_decoder",
        interpret=interpret,
    )(t2, node_latent, w1b, b1r, w2b, b2r, wp, gcb)
